# Initial kernel scaffold; baseline (speedup 1.0000x reference)
import jax
import jax.numpy as jnp
from jax.experimental import pallas as pl


def kernel(x, perm, perm_inv):
    raise NotImplementedError("write your pallas kernel here")


# speedup vs baseline: 4.8561x; 4.8561x over previous
import jax
import jax.numpy as jnp
from jax.experimental import pallas as pl


_ROWS, _COLS = 16384, 1024
_BLK = 1024


def _permute_body(perm_ref, x_ref, o_ref):
    p = perm_ref[...]
    k = jax.lax.broadcasted_iota(jnp.int32, (_COLS, _COLS), 0)
    onehot = (k == p).astype(jnp.float32)
    o_ref[...] = jnp.dot(x_ref[...], onehot, preferred_element_type=jnp.float32)


def kernel(x, perm, perm_inv):
    del perm_inv
    perm2d = perm.reshape(1, _COLS)
    out = pl.pallas_call(
        _permute_body,
        grid=(_ROWS // _BLK,),
        in_specs=[
            pl.BlockSpec((1, _COLS), lambda i: (0, 0)),
            pl.BlockSpec((_BLK, _COLS), lambda i: (i, 0)),
        ],
        out_specs=pl.BlockSpec((_BLK, _COLS), lambda i: (i, 0)),
        out_shape=jax.ShapeDtypeStruct((_ROWS, _COLS), jnp.float32),
    )(perm2d, x)
    return out
